# baseline (speedup 1.0000x reference)
import functools

import jax
import jax.numpy as jnp
from jax import lax
from jax.experimental import pallas as pl
from jax.experimental.pallas import tpu as pltpu
from jax.experimental.pallas import tpu_sc as plsc

Q = 1024
D = 64
KN = 100000
BLK = 128
NB = 784
KP = NB * BLK
CHUNK = 2048
NCHUNK = KP // CHUNK
BPC = CHUNK // BLK
NSEL = 40
CAND = NSEL * BLK
TOPK = 32

NWAVES = 2
QW = Q // NWAVES

NC, NS, L = 2, 16, 16
NW = NC * NS
QPW = QW // NW
RPW = (QW * TOPK) // NW

_RCHUNK = 128
_ROWS_W = QPW * NSEL
_NRCH = _ROWS_W // _RCHUNK
_NKCH = RPW // _RCHUNK
_NBUF = 4

NEG_INF = float("-inf")


def _score_body(q_ref, k_ref, q2_ref, k2_ref, s_ref, bm_ref):
    i = pl.program_id(0)
    q = q_ref[...]
    kc = k_ref[...]
    dots = lax.dot_general(q, kc, (((1,), (1,)), ((), ())),
                           preferred_element_type=jnp.float32)
    q2 = q2_ref[...]
    k2 = k2_ref[...]
    s = 2.0 * dots - q2 - k2
    col = i * CHUNK + lax.broadcasted_iota(jnp.int32, (1, CHUNK), 1)
    s = jnp.where(col < KN, s, NEG_INF)
    s_ref[...] = s
    parts = [jnp.max(s[:, j * BLK:(j + 1) * BLK], axis=1, keepdims=True)
             for j in range(BPC)]
    bm_ref[...] = jnp.concatenate(parts, axis=1)[None]


_score_call = pl.pallas_call(
    _score_body,
    grid=(NCHUNK,),
    in_specs=[
        pl.BlockSpec((Q, D), lambda i: (0, 0)),
        pl.BlockSpec((CHUNK, D), lambda i: (i, 0)),
        pl.BlockSpec((Q, 1), lambda i: (0, 0)),
        pl.BlockSpec((1, CHUNK), lambda i: (0, i)),
    ],
    out_specs=[
        pl.BlockSpec((Q, CHUNK), lambda i: (0, i)),
        pl.BlockSpec((1, Q, BPC), lambda i: (i, 0, 0)),
    ],
    out_shape=[
        jax.ShapeDtypeStruct((Q, KP), jnp.float32),
        jax.ShapeDtypeStruct((NCHUNK, Q, BPC), jnp.float32),
    ],
)


def _select_body(bm_ref, bids_ref):
    bm = bm_ref[...]
    cid = lax.broadcasted_iota(jnp.int32, (QW, NB), 1)
    tcol = lax.broadcasted_iota(jnp.int32, (QW, NSEL), 1)

    def step(t, carry):
        bm, bids = carry
        a = jnp.argmax(bm, axis=1).astype(jnp.int32)
        bids = jnp.where(tcol == t, a[:, None], bids)
        bm = jnp.where(cid == a[:, None], NEG_INF, bm)
        return bm, bids

    bids0 = jnp.zeros((QW, NSEL), jnp.int32)
    _, bids = lax.fori_loop(0, NSEL, step, (bm, bids0))

    def sort_step(t, carry):
        bb, sb = carry
        mn = jnp.min(bb, axis=1)
        sb = jnp.where(tcol == t, mn[:, None], sb)
        bb = jnp.where(bb == mn[:, None], NB + 1, bb)
        return bb, sb

    _, sbids = lax.fori_loop(0, NSEL, sort_step,
                             (bids, jnp.zeros((QW, NSEL), jnp.int32)))
    bids_ref[...] = sbids


_select_call = pl.pallas_call(
    _select_body,
    out_shape=jax.ShapeDtypeStruct((QW, NSEL), jnp.int32),
)


def _sc_compact_body(scores_hbm, fids_hbm, cand_hbm,
                     fidv, cv0, cv1, cv2, cv3, sem0, sem1, sem2, sem3):
    wid = lax.axis_index("s") * NC + lax.axis_index("c")
    pltpu.sync_copy(fids_hbm.at[wid], fidv)

    bufs = (cv0, cv1, cv2, cv3)
    sems = (sem0, sem1, sem2, sem3)

    cps = [
        pltpu.async_copy(scores_hbm.at[fidv.at[c]], bufs[c % _NBUF],
                         sems[c % _NBUF])
        for c in range(min(_NBUF, _NRCH))
    ]
    for c in range(_NRCH):
        cps[c % _NBUF].wait()
        pltpu.sync_copy(
            bufs[c % _NBUF],
            cand_hbm.at[pl.ds(wid * _ROWS_W + c * _RCHUNK, _RCHUNK)])
        if c + _NBUF < _NRCH:
            cps[c % _NBUF] = pltpu.async_copy(
                scores_hbm.at[fidv.at[c + _NBUF]], bufs[c % _NBUF],
                sems[c % _NBUF])


QB = 256


def _final_body(v_ref, b_ref, s_ref, i_ref):
    v = v_ref[...]
    b = b_ref[...]
    iota_c = lax.broadcasted_iota(jnp.int32, (QB, CAND), 1)
    iota_k = lax.broadcasted_iota(jnp.int32, (QB, TOPK), 1)

    def step(t, carry):
        v, sv, sp = carry
        m = jnp.max(v, axis=1)
        a = jnp.argmax(v, axis=1).astype(jnp.int32)
        v = jnp.where(iota_c == a[:, None], NEG_INF, v)
        sel_t = iota_k == t
        sv = jnp.where(sel_t, m[:, None], sv)
        sp = jnp.where(sel_t, a[:, None], sp)
        return v, sv, sp

    sv0 = jnp.zeros((QB, TOPK), jnp.float32)
    sp0 = jnp.zeros((QB, TOPK), jnp.int32)
    _, sv, sp = lax.fori_loop(0, TOPK, step, (v, sv0, sp0))

    blk_j = sp >> 7
    lane = sp & (BLK - 1)
    bj = jnp.sum(jnp.where(blk_j[:, :, None] ==
                           lax.broadcasted_iota(jnp.int32, (QB, TOPK, NSEL), 2),
                           b[:, None, :], 0), axis=2)
    s_ref[...] = sv
    i_ref[...] = bj * BLK + lane


_final_call = pl.pallas_call(
    _final_body,
    grid=(QW // QB,),
    in_specs=[
        pl.BlockSpec((QB, CAND), lambda i: (i, 0)),
        pl.BlockSpec((QB, NSEL), lambda i: (i, 0)),
    ],
    out_specs=[
        pl.BlockSpec((QB, TOPK), lambda i: (i, 0)),
        pl.BlockSpec((QB, TOPK), lambda i: (i, 0)),
    ],
    out_shape=[
        jax.ShapeDtypeStruct((QW, TOPK), jnp.float32),
        jax.ShapeDtypeStruct((QW, TOPK), jnp.int32),
    ],
)


def _sc_gather_body(keys_hbm, idx_hbm, out_hbm, idxv, rows0, rows1,
                    rows2, rows3, sem0, sem1, sem2, sem3):
    wid = lax.axis_index("s") * NC + lax.axis_index("c")
    base = wid * RPW
    pltpu.sync_copy(idx_hbm.at[wid], idxv)

    bufs = (rows0, rows1, rows2, rows3)
    sems = (sem0, sem1, sem2, sem3)
    cps = [
        pltpu.async_copy(keys_hbm.at[idxv.at[c]], bufs[c % _NBUF],
                         sems[c % _NBUF])
        for c in range(min(_NBUF, _NKCH))
    ]
    for c in range(_NKCH):
        cps[c % _NBUF].wait()
        pltpu.sync_copy(bufs[c % _NBUF],
                        out_hbm.at[pl.ds(base + c * _RCHUNK, _RCHUNK)])
        if c + _NBUF < _NKCH:
            cps[c % _NBUF] = pltpu.async_copy(
                keys_hbm.at[idxv.at[c + _NBUF]], bufs[c % _NBUF],
                sems[c % _NBUF])


def _out_body(s_ref, g_ref, o_ref):
    s = s_ref[...]
    g = g_ref[...][:, :, :D]
    mx = jnp.max(s, axis=1, keepdims=True)
    e = jnp.exp(s - mx)
    w = e / jnp.sum(e, axis=1, keepdims=True)
    o_ref[...] = jnp.sum(w[:, :, None] * g, axis=1)


_out_call = pl.pallas_call(
    _out_body,
    out_shape=jax.ShapeDtypeStruct((QW, D), jnp.float32),
)


@functools.lru_cache(maxsize=1)
def _sc_calls():
    mesh = plsc.VectorSubcoreMesh(core_axis_name="c", subcore_axis_name="s")
    compact = pl.kernel(
        _sc_compact_body,
        mesh=mesh,
        out_type=jax.ShapeDtypeStruct((QW * NSEL, BLK), jnp.float32),
        scratch_types=[
            pltpu.VMEM((_NRCH, _RCHUNK), jnp.int32),
            pltpu.VMEM((_RCHUNK, BLK), jnp.float32),
            pltpu.VMEM((_RCHUNK, BLK), jnp.float32),
            pltpu.VMEM((_RCHUNK, BLK), jnp.float32),
            pltpu.VMEM((_RCHUNK, BLK), jnp.float32),
            pltpu.SemaphoreType.DMA,
            pltpu.SemaphoreType.DMA,
            pltpu.SemaphoreType.DMA,
            pltpu.SemaphoreType.DMA,
        ],
    )
    gather = pl.kernel(
        _sc_gather_body,
        mesh=mesh,
        out_type=jax.ShapeDtypeStruct((QW * TOPK, 2 * D), jnp.float32),
        scratch_types=[
            pltpu.VMEM((_NKCH, _RCHUNK), jnp.int32),
            pltpu.VMEM((_RCHUNK, 2 * D), jnp.float32),
            pltpu.VMEM((_RCHUNK, 2 * D), jnp.float32),
            pltpu.VMEM((_RCHUNK, 2 * D), jnp.float32),
            pltpu.VMEM((_RCHUNK, 2 * D), jnp.float32),
            pltpu.SemaphoreType.DMA,
            pltpu.SemaphoreType.DMA,
            pltpu.SemaphoreType.DMA,
            pltpu.SemaphoreType.DMA,
        ],
    )
    return compact, gather


def kernel(queries, keys, k):
    del k
    sc_compact, sc_gather = _sc_calls()
    keys_p = jnp.pad(keys, ((0, KP - KN), (0, 0)))
    q2 = jnp.sum(queries * queries, axis=-1, keepdims=True)
    k2 = jnp.pad(jnp.sum(keys * keys, axis=-1), (0, KP - KN))
    scores, bmax3 = _score_call(queries, keys_p, q2, k2[None, :])
    bmax = jnp.transpose(bmax3, (1, 0, 2)).reshape(Q, NB)
    scores2 = scores.reshape(Q * NB, BLK)
    keys_w = jnp.pad(keys, ((0, 0), (0, D)))

    outs, tvs, tis = [], [], []
    for w in range(NWAVES):
        sb = _select_call(bmax[w * QW:(w + 1) * QW])
        fids = sb + (jnp.arange(QW, dtype=jnp.int32)[:, None]
                     + w * QW) * NB
        cand = sc_compact(scores2, fids.reshape(NW, _NRCH, _RCHUNK))
        topv, topidx = _final_call(cand.reshape(QW, CAND), sb)
        gk = sc_gather(keys_w, topidx.reshape(NW, _NKCH, _RCHUNK))
        out = _out_call(topv, gk.reshape(QW, TOPK, 2 * D))
        outs.append(out)
        tvs.append(topv)
        tis.append(topidx)

    return (jnp.concatenate(outs, axis=0),
            jnp.concatenate(tvs, axis=0),
            jnp.concatenate(tis, axis=0))


# speedup vs baseline: 1.0224x; 1.0224x over previous
import functools

import jax
import jax.numpy as jnp
from jax import lax
from jax.experimental import pallas as pl
from jax.experimental.pallas import tpu as pltpu
from jax.experimental.pallas import tpu_sc as plsc

Q = 1024
D = 64
KN = 100000
BLK = 128
NB = 800
KP = NB * BLK
CHUNK = 4096
NCHUNK = KP // CHUNK
BPC = CHUNK // BLK
NSEL = 40
CAND = NSEL * BLK
TOPK = 32

NWAVES = 2
QW = Q // NWAVES

NC, NS, L = 2, 16, 16
NW = NC * NS
QPW = QW // NW
RPW = (QW * TOPK) // NW

_RCHUNK = 128
_ROWS_W = QPW * NSEL
_NRCH = _ROWS_W // _RCHUNK
_NKCH = RPW // _RCHUNK
_NBUF = 4

NEG_INF = float("-inf")


def _score_body(q_ref, k_ref, q2_ref, k2_ref, s_ref, bm_ref):
    i = pl.program_id(0)
    q = q_ref[...]
    kc = k_ref[...]
    dots = lax.dot_general(q, kc, (((1,), (1,)), ((), ())),
                           preferred_element_type=jnp.float32)
    q2 = q2_ref[...]
    k2 = k2_ref[...]
    s = 2.0 * dots - q2 - k2
    col = i * CHUNK + lax.broadcasted_iota(jnp.int32, (1, CHUNK), 1)
    s = jnp.where(col < KN, s, NEG_INF)
    s_ref[...] = s
    parts = [jnp.max(s[:, j * BLK:(j + 1) * BLK], axis=1, keepdims=True)
             for j in range(BPC)]
    bm_ref[...] = jnp.concatenate(parts, axis=1)[None]


_score_call = pl.pallas_call(
    _score_body,
    grid=(NCHUNK,),
    in_specs=[
        pl.BlockSpec((Q, D), lambda i: (0, 0)),
        pl.BlockSpec((CHUNK, D), lambda i: (i, 0)),
        pl.BlockSpec((Q, 1), lambda i: (0, 0)),
        pl.BlockSpec((1, CHUNK), lambda i: (0, i)),
    ],
    out_specs=[
        pl.BlockSpec((Q, CHUNK), lambda i: (0, i)),
        pl.BlockSpec((1, Q, BPC), lambda i: (i, 0, 0)),
    ],
    out_shape=[
        jax.ShapeDtypeStruct((Q, KP), jnp.float32),
        jax.ShapeDtypeStruct((NCHUNK, Q, BPC), jnp.float32),
    ],
)


def _select_body(bm_ref, bids_ref):
    bm = bm_ref[...]
    cid = lax.broadcasted_iota(jnp.int32, (QW, NB), 1)
    tcol = lax.broadcasted_iota(jnp.int32, (QW, NSEL), 1)

    def step(t, carry):
        bm, bids = carry
        m = jnp.max(bm, axis=1)
        eq = bm == m[:, None]
        a = jnp.min(jnp.where(eq, cid, NB), axis=1)
        bids = jnp.where(tcol == t, a[:, None], bids)
        bm = jnp.where(cid == a[:, None], NEG_INF, bm)
        return bm, bids

    bids0 = jnp.zeros((QW, NSEL), jnp.int32)
    _, bids = lax.fori_loop(0, NSEL, step, (bm, bids0))

    def sort_step(t, carry):
        bb, sb = carry
        mn = jnp.min(bb, axis=1)
        sb = jnp.where(tcol == t, mn[:, None], sb)
        bb = jnp.where(bb == mn[:, None], NB + 1, bb)
        return bb, sb

    _, sbids = lax.fori_loop(0, NSEL, sort_step,
                             (bids, jnp.zeros((QW, NSEL), jnp.int32)))
    bids_ref[...] = sbids


_select_call = pl.pallas_call(
    _select_body,
    out_shape=jax.ShapeDtypeStruct((QW, NSEL), jnp.int32),
)


def _sc_compact_body(scores_hbm, fids_hbm, cand_hbm,
                     fidv, cv0, cv1, cv2, cv3, sem0, sem1, sem2, sem3):
    wid = lax.axis_index("s") * NC + lax.axis_index("c")
    pltpu.sync_copy(fids_hbm.at[wid], fidv)

    bufs = (cv0, cv1, cv2, cv3)
    sems = (sem0, sem1, sem2, sem3)

    cps = [
        pltpu.async_copy(scores_hbm.at[fidv.at[c]], bufs[c % _NBUF],
                         sems[c % _NBUF])
        for c in range(min(_NBUF, _NRCH))
    ]
    for c in range(_NRCH):
        cps[c % _NBUF].wait()
        pltpu.sync_copy(
            bufs[c % _NBUF],
            cand_hbm.at[pl.ds(wid * _ROWS_W + c * _RCHUNK, _RCHUNK)])
        if c + _NBUF < _NRCH:
            cps[c % _NBUF] = pltpu.async_copy(
                scores_hbm.at[fidv.at[c + _NBUF]], bufs[c % _NBUF],
                sems[c % _NBUF])


QB = 256


def _final_body(v_ref, b_ref, s_ref, i_ref):
    v = v_ref[...]
    b = b_ref[...]
    iota_c = lax.broadcasted_iota(jnp.int32, (QB, CAND), 1)
    iota_k = lax.broadcasted_iota(jnp.int32, (QB, TOPK), 1)

    def step(t, carry):
        v, sv, sp = carry
        m = jnp.max(v, axis=1)
        eq = v == m[:, None]
        a = jnp.min(jnp.where(eq, iota_c, CAND), axis=1)
        v = jnp.where(iota_c == a[:, None], NEG_INF, v)
        sel_t = iota_k == t
        sv = jnp.where(sel_t, m[:, None], sv)
        sp = jnp.where(sel_t, a[:, None], sp)
        return v, sv, sp

    sv0 = jnp.zeros((QB, TOPK), jnp.float32)
    sp0 = jnp.zeros((QB, TOPK), jnp.int32)
    _, sv, sp = lax.fori_loop(0, TOPK, step, (v, sv0, sp0))

    blk_j = sp >> 7
    lane = sp & (BLK - 1)
    bj = jnp.sum(jnp.where(blk_j[:, :, None] ==
                           lax.broadcasted_iota(jnp.int32, (QB, TOPK, NSEL), 2),
                           b[:, None, :], 0), axis=2)
    s_ref[...] = sv
    i_ref[...] = bj * BLK + lane


_final_call = pl.pallas_call(
    _final_body,
    grid=(QW // QB,),
    in_specs=[
        pl.BlockSpec((QB, CAND), lambda i: (i, 0)),
        pl.BlockSpec((QB, NSEL), lambda i: (i, 0)),
    ],
    out_specs=[
        pl.BlockSpec((QB, TOPK), lambda i: (i, 0)),
        pl.BlockSpec((QB, TOPK), lambda i: (i, 0)),
    ],
    out_shape=[
        jax.ShapeDtypeStruct((QW, TOPK), jnp.float32),
        jax.ShapeDtypeStruct((QW, TOPK), jnp.int32),
    ],
)


def _sc_gather_body(keys_hbm, idx_hbm, out_hbm, idxv, rows0, rows1,
                    rows2, rows3, sem0, sem1, sem2, sem3):
    wid = lax.axis_index("s") * NC + lax.axis_index("c")
    base = wid * RPW
    pltpu.sync_copy(idx_hbm.at[wid], idxv)

    bufs = (rows0, rows1, rows2, rows3)
    sems = (sem0, sem1, sem2, sem3)
    cps = [
        pltpu.async_copy(keys_hbm.at[idxv.at[c]], bufs[c % _NBUF],
                         sems[c % _NBUF])
        for c in range(min(_NBUF, _NKCH))
    ]
    for c in range(_NKCH):
        cps[c % _NBUF].wait()
        pltpu.sync_copy(bufs[c % _NBUF],
                        out_hbm.at[pl.ds(base + c * _RCHUNK, _RCHUNK)])
        if c + _NBUF < _NKCH:
            cps[c % _NBUF] = pltpu.async_copy(
                keys_hbm.at[idxv.at[c + _NBUF]], bufs[c % _NBUF],
                sems[c % _NBUF])


def _out_body(s_ref, g_ref, o_ref):
    s = s_ref[...]
    g = g_ref[...][:, :, :D]
    mx = jnp.max(s, axis=1, keepdims=True)
    e = jnp.exp(s - mx)
    w = e / jnp.sum(e, axis=1, keepdims=True)
    o_ref[...] = jnp.sum(w[:, :, None] * g, axis=1)


_out_call = pl.pallas_call(
    _out_body,
    out_shape=jax.ShapeDtypeStruct((QW, D), jnp.float32),
)


@functools.lru_cache(maxsize=1)
def _sc_calls():
    mesh = plsc.VectorSubcoreMesh(core_axis_name="c", subcore_axis_name="s")
    compact = pl.kernel(
        _sc_compact_body,
        mesh=mesh,
        out_type=jax.ShapeDtypeStruct((QW * NSEL, BLK), jnp.float32),
        scratch_types=[
            pltpu.VMEM((_NRCH, _RCHUNK), jnp.int32),
            pltpu.VMEM((_RCHUNK, BLK), jnp.float32),
            pltpu.VMEM((_RCHUNK, BLK), jnp.float32),
            pltpu.VMEM((_RCHUNK, BLK), jnp.float32),
            pltpu.VMEM((_RCHUNK, BLK), jnp.float32),
            pltpu.SemaphoreType.DMA,
            pltpu.SemaphoreType.DMA,
            pltpu.SemaphoreType.DMA,
            pltpu.SemaphoreType.DMA,
        ],
    )
    gather = pl.kernel(
        _sc_gather_body,
        mesh=mesh,
        out_type=jax.ShapeDtypeStruct((QW * TOPK, 2 * D), jnp.float32),
        scratch_types=[
            pltpu.VMEM((_NKCH, _RCHUNK), jnp.int32),
            pltpu.VMEM((_RCHUNK, 2 * D), jnp.float32),
            pltpu.VMEM((_RCHUNK, 2 * D), jnp.float32),
            pltpu.VMEM((_RCHUNK, 2 * D), jnp.float32),
            pltpu.VMEM((_RCHUNK, 2 * D), jnp.float32),
            pltpu.SemaphoreType.DMA,
            pltpu.SemaphoreType.DMA,
            pltpu.SemaphoreType.DMA,
            pltpu.SemaphoreType.DMA,
        ],
    )
    return compact, gather


def kernel(queries, keys, k):
    del k
    sc_compact, sc_gather = _sc_calls()
    keys_p = jnp.pad(keys, ((0, KP - KN), (0, 0)))
    q2 = jnp.sum(queries * queries, axis=-1, keepdims=True)
    k2 = jnp.pad(jnp.sum(keys * keys, axis=-1), (0, KP - KN))
    scores, bmax3 = _score_call(queries, keys_p, q2, k2[None, :])
    bmax = jnp.transpose(bmax3, (1, 0, 2)).reshape(Q, NB)
    scores2 = scores.reshape(Q * NB, BLK)
    keys_w = jnp.pad(keys, ((0, 0), (0, D)))

    outs, tvs, tis = [], [], []
    for w in range(NWAVES):
        sb = _select_call(bmax[w * QW:(w + 1) * QW])
        fids = sb + (jnp.arange(QW, dtype=jnp.int32)[:, None]
                     + w * QW) * NB
        cand = sc_compact(scores2, fids.reshape(NW, _NRCH, _RCHUNK))
        topv, topidx = _final_call(cand.reshape(QW, CAND), sb)
        gk = sc_gather(keys_w, topidx.reshape(NW, _NKCH, _RCHUNK))
        out = _out_call(topv, gk.reshape(QW, TOPK, 2 * D))
        outs.append(out)
        tvs.append(topv)
        tis.append(topidx)

    return (jnp.concatenate(outs, axis=0),
            jnp.concatenate(tvs, axis=0),
            jnp.concatenate(tis, axis=0))


# speedup vs baseline: 1.0768x; 1.0532x over previous
import functools

import jax
import jax.numpy as jnp
from jax import lax
from jax.experimental import pallas as pl
from jax.experimental.pallas import tpu as pltpu
from jax.experimental.pallas import tpu_sc as plsc

Q = 1024
D = 64
KN = 100000
BLK = 128
NB = 784
KP = NB * BLK
CHUNK = 2048
NCHUNK = KP // CHUNK
BPC = CHUNK // BLK
NSEL = 40
CAND = NSEL * BLK
TOPK = 32

NWAVES = 1
QW = Q // NWAVES

NC, NS, L = 2, 16, 16
NW = NC * NS
QPW = QW // NW
RPW = (QW * TOPK) // NW

_RCHUNK = 128
_ROWS_W = QPW * NSEL
_NRCH = _ROWS_W // _RCHUNK
_NKCH = RPW // _RCHUNK
_NBUF = 4

NEG_INF = float("-inf")


def _score_body(q_ref, k_ref, q2_ref, k2_ref, s_ref, bm_ref):
    i = pl.program_id(0)
    q = q_ref[...]
    kc = k_ref[...]
    dots = lax.dot_general(q, kc, (((1,), (1,)), ((), ())),
                           preferred_element_type=jnp.float32)
    q2 = q2_ref[...]
    k2 = k2_ref[...]
    s = 2.0 * dots - q2 - k2
    col = i * CHUNK + lax.broadcasted_iota(jnp.int32, (1, CHUNK), 1)
    s = jnp.where(col < KN, s, NEG_INF)
    s_ref[...] = s
    parts = [jnp.max(s[:, j * BLK:(j + 1) * BLK], axis=1, keepdims=True)
             for j in range(BPC)]
    bm_ref[...] = jnp.concatenate(parts, axis=1)[None]


_score_call = pl.pallas_call(
    _score_body,
    grid=(NCHUNK,),
    in_specs=[
        pl.BlockSpec((Q, D), lambda i: (0, 0)),
        pl.BlockSpec((CHUNK, D), lambda i: (i, 0)),
        pl.BlockSpec((Q, 1), lambda i: (0, 0)),
        pl.BlockSpec((1, CHUNK), lambda i: (0, i)),
    ],
    out_specs=[
        pl.BlockSpec((Q, CHUNK), lambda i: (0, i)),
        pl.BlockSpec((1, Q, BPC), lambda i: (i, 0, 0)),
    ],
    out_shape=[
        jax.ShapeDtypeStruct((Q, KP), jnp.float32),
        jax.ShapeDtypeStruct((NCHUNK, Q, BPC), jnp.float32),
    ],
)


def _select_body(bm_ref, bids_ref):
    bm = bm_ref[...]
    cid = lax.broadcasted_iota(jnp.int32, (QW, NB), 1)
    tcol = lax.broadcasted_iota(jnp.int32, (QW, NSEL), 1)

    def step(t, carry):
        bm, bids = carry
        m = jnp.max(bm, axis=1)
        eq = bm == m[:, None]
        a = jnp.min(jnp.where(eq, cid, NB), axis=1)
        bids = jnp.where(tcol == t, a[:, None], bids)
        bm = jnp.where(cid == a[:, None], NEG_INF, bm)
        return bm, bids

    bids0 = jnp.zeros((QW, NSEL), jnp.int32)
    _, bids = lax.fori_loop(0, NSEL, step, (bm, bids0))

    def sort_step(t, carry):
        bb, sb = carry
        mn = jnp.min(bb, axis=1)
        sb = jnp.where(tcol == t, mn[:, None], sb)
        bb = jnp.where(bb == mn[:, None], NB + 1, bb)
        return bb, sb

    _, sbids = lax.fori_loop(0, NSEL, sort_step,
                             (bids, jnp.zeros((QW, NSEL), jnp.int32)))
    bids_ref[...] = sbids


_select_call = pl.pallas_call(
    _select_body,
    out_shape=jax.ShapeDtypeStruct((QW, NSEL), jnp.int32),
)


def _sc_compact_body(scores_hbm, fids_hbm, cand_hbm,
                     fidv, cv0, cv1, cv2, cv3, sem0, sem1, sem2, sem3):
    wid = lax.axis_index("s") * NC + lax.axis_index("c")
    pltpu.sync_copy(fids_hbm.at[wid], fidv)

    bufs = (cv0, cv1, cv2, cv3)
    sems = (sem0, sem1, sem2, sem3)

    cps = [
        pltpu.async_copy(scores_hbm.at[fidv.at[c]], bufs[c % _NBUF],
                         sems[c % _NBUF])
        for c in range(min(_NBUF, _NRCH))
    ]
    for c in range(_NRCH):
        cps[c % _NBUF].wait()
        pltpu.sync_copy(
            bufs[c % _NBUF],
            cand_hbm.at[pl.ds(wid * _ROWS_W + c * _RCHUNK, _RCHUNK)])
        if c + _NBUF < _NRCH:
            cps[c % _NBUF] = pltpu.async_copy(
                scores_hbm.at[fidv.at[c + _NBUF]], bufs[c % _NBUF],
                sems[c % _NBUF])


QB = 256


def _final_body(v_ref, b_ref, s_ref, i_ref):
    v = v_ref[...]
    b = b_ref[...]
    iota_c = lax.broadcasted_iota(jnp.int32, (QB, CAND), 1)
    iota_k = lax.broadcasted_iota(jnp.int32, (QB, TOPK), 1)

    def step(t, carry):
        v, sv, sp = carry
        m = jnp.max(v, axis=1)
        eq = v == m[:, None]
        a = jnp.min(jnp.where(eq, iota_c, CAND), axis=1)
        v = jnp.where(iota_c == a[:, None], NEG_INF, v)
        sel_t = iota_k == t
        sv = jnp.where(sel_t, m[:, None], sv)
        sp = jnp.where(sel_t, a[:, None], sp)
        return v, sv, sp

    sv0 = jnp.zeros((QB, TOPK), jnp.float32)
    sp0 = jnp.zeros((QB, TOPK), jnp.int32)
    _, sv, sp = lax.fori_loop(0, TOPK, step, (v, sv0, sp0))

    blk_j = sp >> 7
    lane = sp & (BLK - 1)
    bj = jnp.sum(jnp.where(blk_j[:, :, None] ==
                           lax.broadcasted_iota(jnp.int32, (QB, TOPK, NSEL), 2),
                           b[:, None, :], 0), axis=2)
    s_ref[...] = sv
    i_ref[...] = bj * BLK + lane


_final_call = pl.pallas_call(
    _final_body,
    grid=(QW // QB,),
    in_specs=[
        pl.BlockSpec((QB, CAND), lambda i: (i, 0)),
        pl.BlockSpec((QB, NSEL), lambda i: (i, 0)),
    ],
    out_specs=[
        pl.BlockSpec((QB, TOPK), lambda i: (i, 0)),
        pl.BlockSpec((QB, TOPK), lambda i: (i, 0)),
    ],
    out_shape=[
        jax.ShapeDtypeStruct((QW, TOPK), jnp.float32),
        jax.ShapeDtypeStruct((QW, TOPK), jnp.int32),
    ],
)


def _sc_gather_body(keys_hbm, idx_hbm, out_hbm, idxv, rows0, rows1,
                    rows2, rows3, sem0, sem1, sem2, sem3):
    wid = lax.axis_index("s") * NC + lax.axis_index("c")
    base = wid * RPW
    pltpu.sync_copy(idx_hbm.at[wid], idxv)

    bufs = (rows0, rows1, rows2, rows3)
    sems = (sem0, sem1, sem2, sem3)
    cps = [
        pltpu.async_copy(keys_hbm.at[idxv.at[c]], bufs[c % _NBUF],
                         sems[c % _NBUF])
        for c in range(min(_NBUF, _NKCH))
    ]
    for c in range(_NKCH):
        cps[c % _NBUF].wait()
        pltpu.sync_copy(bufs[c % _NBUF],
                        out_hbm.at[pl.ds(base + c * _RCHUNK, _RCHUNK)])
        if c + _NBUF < _NKCH:
            cps[c % _NBUF] = pltpu.async_copy(
                keys_hbm.at[idxv.at[c + _NBUF]], bufs[c % _NBUF],
                sems[c % _NBUF])


def _out_body(s_ref, g_ref, o_ref):
    s = s_ref[...]
    g = g_ref[...][:, :, :D]
    mx = jnp.max(s, axis=1, keepdims=True)
    e = jnp.exp(s - mx)
    w = e / jnp.sum(e, axis=1, keepdims=True)
    o_ref[...] = jnp.sum(w[:, :, None] * g, axis=1)


_out_call = pl.pallas_call(
    _out_body,
    out_shape=jax.ShapeDtypeStruct((QW, D), jnp.float32),
)


@functools.lru_cache(maxsize=1)
def _sc_calls():
    mesh = plsc.VectorSubcoreMesh(core_axis_name="c", subcore_axis_name="s")
    compact = pl.kernel(
        _sc_compact_body,
        mesh=mesh,
        out_type=jax.ShapeDtypeStruct((QW * NSEL, BLK), jnp.float32),
        scratch_types=[
            pltpu.VMEM((_NRCH, _RCHUNK), jnp.int32),
            pltpu.VMEM((_RCHUNK, BLK), jnp.float32),
            pltpu.VMEM((_RCHUNK, BLK), jnp.float32),
            pltpu.VMEM((_RCHUNK, BLK), jnp.float32),
            pltpu.VMEM((_RCHUNK, BLK), jnp.float32),
            pltpu.SemaphoreType.DMA,
            pltpu.SemaphoreType.DMA,
            pltpu.SemaphoreType.DMA,
            pltpu.SemaphoreType.DMA,
        ],
    )
    gather = pl.kernel(
        _sc_gather_body,
        mesh=mesh,
        out_type=jax.ShapeDtypeStruct((QW * TOPK, 2 * D), jnp.float32),
        scratch_types=[
            pltpu.VMEM((_NKCH, _RCHUNK), jnp.int32),
            pltpu.VMEM((_RCHUNK, 2 * D), jnp.float32),
            pltpu.VMEM((_RCHUNK, 2 * D), jnp.float32),
            pltpu.VMEM((_RCHUNK, 2 * D), jnp.float32),
            pltpu.VMEM((_RCHUNK, 2 * D), jnp.float32),
            pltpu.SemaphoreType.DMA,
            pltpu.SemaphoreType.DMA,
            pltpu.SemaphoreType.DMA,
            pltpu.SemaphoreType.DMA,
        ],
    )
    return compact, gather


def kernel(queries, keys, k):
    del k
    sc_compact, sc_gather = _sc_calls()
    keys_p = jnp.pad(keys, ((0, KP - KN), (0, 0)))
    q2 = jnp.sum(queries * queries, axis=-1, keepdims=True)
    k2 = jnp.pad(jnp.sum(keys * keys, axis=-1), (0, KP - KN))
    scores, bmax3 = _score_call(queries, keys_p, q2, k2[None, :])
    bmax = jnp.transpose(bmax3, (1, 0, 2)).reshape(Q, NB)
    scores2 = scores.reshape(Q * NB, BLK)
    keys_w = jnp.pad(keys, ((0, 0), (0, D)))

    outs, tvs, tis = [], [], []
    for w in range(NWAVES):
        sb = _select_call(bmax[w * QW:(w + 1) * QW])
        fids = sb + (jnp.arange(QW, dtype=jnp.int32)[:, None]
                     + w * QW) * NB
        cand = sc_compact(scores2, fids.reshape(NW, _NRCH, _RCHUNK))
        topv, topidx = _final_call(cand.reshape(QW, CAND), sb)
        gk = sc_gather(keys_w, topidx.reshape(NW, _NKCH, _RCHUNK))
        out = _out_call(topv, gk.reshape(QW, TOPK, 2 * D))
        outs.append(out)
        tvs.append(topv)
        tis.append(topidx)

    return (jnp.concatenate(outs, axis=0),
            jnp.concatenate(tvs, axis=0),
            jnp.concatenate(tis, axis=0))


# speedup vs baseline: 1.0901x; 1.0124x over previous
import functools

import jax
import jax.numpy as jnp
from jax import lax
from jax.experimental import pallas as pl
from jax.experimental.pallas import tpu as pltpu
from jax.experimental.pallas import tpu_sc as plsc

Q = 1024
D = 64
KN = 100000
BLK = 128
NB = 784
KP = NB * BLK
CHUNK = 2048
NCHUNK = KP // CHUNK
BPC = CHUNK // BLK
NSEL = 40
CAND = NSEL * BLK
TOPK = 32

NWAVES = 1
QW = Q // NWAVES

NC, NS, L = 2, 16, 16
NW = NC * NS
QPW = QW // NW
RPW = (QW * TOPK) // NW

_RCHUNK = 128
_ROWS_W = QPW * NSEL
_NRCH = _ROWS_W // _RCHUNK
_NKCH = RPW // _RCHUNK
_NBUF = 4

NEG_INF = float("-inf")


def _score_body(q_ref, k_ref, q2_ref, k2_ref, s_ref, bm_ref):
    i = pl.program_id(0)
    q = q_ref[...]
    kc = k_ref[...]
    dots = lax.dot_general(q, kc, (((1,), (1,)), ((), ())),
                           preferred_element_type=jnp.float32)
    q2 = q2_ref[...]
    k2 = k2_ref[...]
    s = 2.0 * dots - q2 - k2
    col = i * CHUNK + lax.broadcasted_iota(jnp.int32, (1, CHUNK), 1)
    s = jnp.where(col < KN, s, NEG_INF)
    s_ref[...] = s
    parts = [jnp.max(s[:, j * BLK:(j + 1) * BLK], axis=1, keepdims=True)
             for j in range(BPC)]
    bm_ref[...] = jnp.concatenate(parts, axis=1)[None]


_score_call = pl.pallas_call(
    _score_body,
    grid=(NCHUNK,),
    in_specs=[
        pl.BlockSpec((Q, D), lambda i: (0, 0)),
        pl.BlockSpec((CHUNK, D), lambda i: (i, 0)),
        pl.BlockSpec((Q, 1), lambda i: (0, 0)),
        pl.BlockSpec((1, CHUNK), lambda i: (0, i)),
    ],
    out_specs=[
        pl.BlockSpec((Q, CHUNK), lambda i: (0, i)),
        pl.BlockSpec((1, Q, BPC), lambda i: (i, 0, 0)),
    ],
    out_shape=[
        jax.ShapeDtypeStruct((Q, KP), jnp.float32),
        jax.ShapeDtypeStruct((NCHUNK, Q, BPC), jnp.float32),
    ],
)


def _select_body(bm_ref, bids_ref):
    bm = bm_ref[...]
    cid = lax.broadcasted_iota(jnp.int32, (QW, NB), 1)
    tcol = lax.broadcasted_iota(jnp.int32, (QW, NSEL), 1)

    def step(t, carry):
        bm, bids = carry
        m = jnp.max(bm, axis=1)
        eq = bm == m[:, None]
        a = jnp.min(jnp.where(eq, cid, NB), axis=1)
        bids = jnp.where(tcol == t, a[:, None], bids)
        bm = jnp.where(cid == a[:, None], NEG_INF, bm)
        return bm, bids

    bids0 = jnp.zeros((QW, NSEL), jnp.int32)
    _, bids = lax.fori_loop(0, NSEL, step, (bm, bids0))

    def sort_step(t, carry):
        bb, sb = carry
        mn = jnp.min(bb, axis=1)
        sb = jnp.where(tcol == t, mn[:, None], sb)
        bb = jnp.where(bb == mn[:, None], NB + 1, bb)
        return bb, sb

    _, sbids = lax.fori_loop(0, NSEL, sort_step,
                             (bids, jnp.zeros((QW, NSEL), jnp.int32)))
    bids_ref[...] = sbids


_select_call = pl.pallas_call(
    _select_body,
    out_shape=jax.ShapeDtypeStruct((QW, NSEL), jnp.int32),
)


def _sc_compact_body(scores_hbm, fids_hbm, cand_hbm,
                     fidv, cv0, cv1, cv2, cv3, sem0, sem1, sem2, sem3):
    wid = lax.axis_index("s") * NC + lax.axis_index("c")
    pltpu.sync_copy(fids_hbm.at[wid], fidv)

    bufs = (cv0, cv1, cv2, cv3)
    sems = (sem0, sem1, sem2, sem3)

    cps = [
        pltpu.async_copy(scores_hbm.at[fidv.at[c]], bufs[c % _NBUF],
                         sems[c % _NBUF])
        for c in range(min(_NBUF, _NRCH))
    ]
    for c in range(_NRCH):
        cps[c % _NBUF].wait()
        pltpu.sync_copy(
            bufs[c % _NBUF],
            cand_hbm.at[pl.ds(wid * _ROWS_W + c * _RCHUNK, _RCHUNK)])
        if c + _NBUF < _NRCH:
            cps[c % _NBUF] = pltpu.async_copy(
                scores_hbm.at[fidv.at[c + _NBUF]], bufs[c % _NBUF],
                sems[c % _NBUF])


QB = 512


def _final_body(v_ref, b_ref, s_ref, i_ref):
    v = v_ref[...]
    b = b_ref[...]
    iota_c = lax.broadcasted_iota(jnp.int32, (QB, CAND), 1)
    iota_k = lax.broadcasted_iota(jnp.int32, (QB, TOPK), 1)

    def step(t, carry):
        v, sv, sp = carry
        m = jnp.max(v, axis=1)
        eq = v == m[:, None]
        a = jnp.min(jnp.where(eq, iota_c, CAND), axis=1)
        v = jnp.where(iota_c == a[:, None], NEG_INF, v)
        sel_t = iota_k == t
        sv = jnp.where(sel_t, m[:, None], sv)
        sp = jnp.where(sel_t, a[:, None], sp)
        return v, sv, sp

    sv0 = jnp.zeros((QB, TOPK), jnp.float32)
    sp0 = jnp.zeros((QB, TOPK), jnp.int32)
    _, sv, sp = lax.fori_loop(0, TOPK, step, (v, sv0, sp0))

    blk_j = sp >> 7
    lane = sp & (BLK - 1)
    bj = jnp.sum(jnp.where(blk_j[:, :, None] ==
                           lax.broadcasted_iota(jnp.int32, (QB, TOPK, NSEL), 2),
                           b[:, None, :], 0), axis=2)
    s_ref[...] = sv
    i_ref[...] = bj * BLK + lane


_final_call = pl.pallas_call(
    _final_body,
    grid=(QW // QB,),
    in_specs=[
        pl.BlockSpec((QB, CAND), lambda i: (i, 0)),
        pl.BlockSpec((QB, NSEL), lambda i: (i, 0)),
    ],
    out_specs=[
        pl.BlockSpec((QB, TOPK), lambda i: (i, 0)),
        pl.BlockSpec((QB, TOPK), lambda i: (i, 0)),
    ],
    out_shape=[
        jax.ShapeDtypeStruct((QW, TOPK), jnp.float32),
        jax.ShapeDtypeStruct((QW, TOPK), jnp.int32),
    ],
)


def _sc_gather_body(keys_hbm, idx_hbm, out_hbm, idxv, rows0, rows1,
                    rows2, rows3, sem0, sem1, sem2, sem3):
    wid = lax.axis_index("s") * NC + lax.axis_index("c")
    base = wid * RPW
    pltpu.sync_copy(idx_hbm.at[wid], idxv)

    bufs = (rows0, rows1, rows2, rows3)
    sems = (sem0, sem1, sem2, sem3)
    cps = [
        pltpu.async_copy(keys_hbm.at[idxv.at[c]], bufs[c % _NBUF],
                         sems[c % _NBUF])
        for c in range(min(_NBUF, _NKCH))
    ]
    for c in range(_NKCH):
        cps[c % _NBUF].wait()
        pltpu.sync_copy(bufs[c % _NBUF],
                        out_hbm.at[pl.ds(base + c * _RCHUNK, _RCHUNK)])
        if c + _NBUF < _NKCH:
            cps[c % _NBUF] = pltpu.async_copy(
                keys_hbm.at[idxv.at[c + _NBUF]], bufs[c % _NBUF],
                sems[c % _NBUF])


def _out_body(s_ref, g_ref, o_ref):
    s = s_ref[...]
    g = g_ref[...][:, :, :D]
    mx = jnp.max(s, axis=1, keepdims=True)
    e = jnp.exp(s - mx)
    w = e / jnp.sum(e, axis=1, keepdims=True)
    o_ref[...] = jnp.sum(w[:, :, None] * g, axis=1)


_out_call = pl.pallas_call(
    _out_body,
    out_shape=jax.ShapeDtypeStruct((QW, D), jnp.float32),
)


@functools.lru_cache(maxsize=1)
def _sc_calls():
    mesh = plsc.VectorSubcoreMesh(core_axis_name="c", subcore_axis_name="s")
    compact = pl.kernel(
        _sc_compact_body,
        mesh=mesh,
        out_type=jax.ShapeDtypeStruct((QW * NSEL, BLK), jnp.float32),
        scratch_types=[
            pltpu.VMEM((_NRCH, _RCHUNK), jnp.int32),
            pltpu.VMEM((_RCHUNK, BLK), jnp.float32),
            pltpu.VMEM((_RCHUNK, BLK), jnp.float32),
            pltpu.VMEM((_RCHUNK, BLK), jnp.float32),
            pltpu.VMEM((_RCHUNK, BLK), jnp.float32),
            pltpu.SemaphoreType.DMA,
            pltpu.SemaphoreType.DMA,
            pltpu.SemaphoreType.DMA,
            pltpu.SemaphoreType.DMA,
        ],
    )
    gather = pl.kernel(
        _sc_gather_body,
        mesh=mesh,
        out_type=jax.ShapeDtypeStruct((QW * TOPK, 2 * D), jnp.float32),
        scratch_types=[
            pltpu.VMEM((_NKCH, _RCHUNK), jnp.int32),
            pltpu.VMEM((_RCHUNK, 2 * D), jnp.float32),
            pltpu.VMEM((_RCHUNK, 2 * D), jnp.float32),
            pltpu.VMEM((_RCHUNK, 2 * D), jnp.float32),
            pltpu.VMEM((_RCHUNK, 2 * D), jnp.float32),
            pltpu.SemaphoreType.DMA,
            pltpu.SemaphoreType.DMA,
            pltpu.SemaphoreType.DMA,
            pltpu.SemaphoreType.DMA,
        ],
    )
    return compact, gather


def kernel(queries, keys, k):
    del k
    sc_compact, sc_gather = _sc_calls()
    keys_p = jnp.pad(keys, ((0, KP - KN), (0, 0)))
    q2 = jnp.sum(queries * queries, axis=-1, keepdims=True)
    k2 = jnp.pad(jnp.sum(keys * keys, axis=-1), (0, KP - KN))
    scores, bmax3 = _score_call(queries, keys_p, q2, k2[None, :])
    bmax = jnp.transpose(bmax3, (1, 0, 2)).reshape(Q, NB)
    scores2 = scores.reshape(Q * NB, BLK)
    keys_w = jnp.pad(keys, ((0, 0), (0, D)))

    outs, tvs, tis = [], [], []
    for w in range(NWAVES):
        sb = _select_call(bmax[w * QW:(w + 1) * QW])
        fids = sb + (jnp.arange(QW, dtype=jnp.int32)[:, None]
                     + w * QW) * NB
        cand = sc_compact(scores2, fids.reshape(NW, _NRCH, _RCHUNK))
        topv, topidx = _final_call(cand.reshape(QW, CAND), sb)
        gk = sc_gather(keys_w, topidx.reshape(NW, _NKCH, _RCHUNK))
        out = _out_call(topv, gk.reshape(QW, TOPK, 2 * D))
        outs.append(out)
        tvs.append(topv)
        tis.append(topidx)

    return (jnp.concatenate(outs, axis=0),
            jnp.concatenate(tvs, axis=0),
            jnp.concatenate(tis, axis=0))
